# baseline (speedup 1.0000x reference)
import functools

import jax
import jax.numpy as jnp
from jax import lax
from jax.experimental import pallas as pl
from jax.experimental.pallas import tpu as pltpu
from jax.experimental.pallas import tpu_sc as plsc

LEAK = 0.1
BE = 6400
NW = 32
_SC_MESH = dict(core_axis_name="c", subcore_axis_name="s")


def _lrelu(h):
    return jnp.where(h > 0, h, LEAK * h)


def _stats_body(v_ref, o_ref):
    i = pl.program_id(0)
    blk = v_ref[...]
    acc = jnp.stack([jnp.sum(blk, axis=0), jnp.sum(blk * blk, axis=0)])

    @pl.when(i == 0)
    def _():
        o_ref[...] = acc

    @pl.when(i > 0)
    def _():
        o_ref[...] += acc


@functools.lru_cache(maxsize=None)
def _make_stats(R, C, BR):
    return pl.pallas_call(
        _stats_body,
        grid=(R // BR,),
        in_specs=[pl.BlockSpec((BR, C), lambda i: (i, 0))],
        out_specs=pl.BlockSpec((2, C), lambda i: (0, 0)),
        out_shape=jax.ShapeDtypeStruct((2, C), jnp.float32),
    )


def _bn_stats(v, BR):
    R, C = v.shape
    st = _make_stats(R, C, BR)(v)
    mu = st[0] / R
    var = st[1] / R - mu * mu
    return jnp.stack([mu, var])


def _bnx_body(v_ref, st_ref, gb_ref, o_ref):
    mu = st_ref[0, :]
    var = st_ref[1, :]
    g = gb_ref[0, :]
    b = gb_ref[1, :]
    o_ref[...] = g * (v_ref[...] - mu) / jnp.sqrt(var + 1e-5) + b


@functools.lru_cache(maxsize=None)
def _make_bnx(N, C):
    return pl.pallas_call(
        _bnx_body,
        out_shape=jax.ShapeDtypeStruct((N, C), jnp.float32),
    )


@functools.lru_cache(maxsize=None)
def _make_pair_gather(N, F, E, W=1000):
    per_w = E // NW
    assert per_w % W == 0 and W % 8 == 0
    nwin = per_w // W
    mesh = plsc.VectorSubcoreMesh(**_SC_MESH)

    @functools.partial(
        pl.kernel,
        mesh=mesh,
        compiler_params=pltpu.CompilerParams(use_tc_tiling_on_sc=False),
        out_type=(
            jax.ShapeDtypeStruct((E, F), jnp.float32),
            jax.ShapeDtypeStruct((E, F), jnp.float32),
        ),
        scratch_types=[
            pltpu.VMEM((W,), jnp.int32),
            pltpu.VMEM((W, F), jnp.float32),
            pltpu.SemaphoreType.DMA,
        ],
    )
    def k(x_hbm, src_hbm, dst_hbm, osrc_hbm, odst_hbm, idx_v, rows_v, sem):
        wid = lax.axis_index("s") * 2 + lax.axis_index("c")
        base = wid * per_w

        def body(i, _):
            off = base + i * W
            pltpu.sync_copy(src_hbm.at[pl.ds(off, W)], idx_v)
            pltpu.async_copy(x_hbm.at[idx_v], rows_v, sem).wait()
            pltpu.sync_copy(rows_v, osrc_hbm.at[pl.ds(off, W)])
            pltpu.sync_copy(dst_hbm.at[pl.ds(off, W)], idx_v)
            pltpu.async_copy(x_hbm.at[idx_v], rows_v, sem).wait()
            pltpu.sync_copy(rows_v, odst_hbm.at[pl.ds(off, W)])
            return ()

        lax.fori_loop(0, nwin, body, (), unroll=False)

    return k


def _pair_gather(x, src, dst):
    N, F = x.shape
    return _make_pair_gather(N, F, src.shape[0])(x, src, dst)


def _mlp3T_body(xi_ref, xj_ref, w0, b0, w1, b1, w2, b2, o_ref):
    xi = xi_ref[...]
    xj = xj_ref[...]
    hcat = jnp.concatenate([xi, xj - xi], axis=1)
    h = jnp.dot(hcat, w0[...].T, preferred_element_type=jnp.float32)
    h = _lrelu(h + b0[...])
    h = _lrelu(jnp.dot(h, w1[...].T, preferred_element_type=jnp.float32) + b1[...])
    m = jnp.dot(h, w2[...].T, preferred_element_type=jnp.float32) + b2[...]
    o_ref[...] = m.T


@functools.lru_cache(maxsize=None)
def _make_mlp3T(E, F, H0, H1, F2):
    specs = [
        pl.BlockSpec((BE, F), lambda i: (i, 0)),
        pl.BlockSpec((BE, F), lambda i: (i, 0)),
        pl.BlockSpec((H0, 2 * F), lambda i: (0, 0)),
        pl.BlockSpec((H0,), lambda i: (0,)),
        pl.BlockSpec((H1, H0), lambda i: (0, 0)),
        pl.BlockSpec((H1,), lambda i: (0,)),
        pl.BlockSpec((F2, H1), lambda i: (0, 0)),
        pl.BlockSpec((F2,), lambda i: (0,)),
    ]
    return pl.pallas_call(
        _mlp3T_body,
        grid=(E // BE,),
        in_specs=specs,
        out_specs=pl.BlockSpec((F2, BE), lambda i: (0, i)),
        out_shape=jax.ShapeDtypeStruct((F2, E), jnp.float32),
    )


@functools.lru_cache(maxsize=None)
def _make_scatter_max(N, F, E, W=2000):
    cpw = -(-F // NW)
    nact = -(-F // cpw)
    nwin = E // W
    nv = W // 16
    mesh = plsc.VectorSubcoreMesh(**_SC_MESH)

    @functools.partial(
        pl.kernel,
        mesh=mesh,
        compiler_params=pltpu.CompilerParams(use_tc_tiling_on_sc=False,
                                             needs_layout_passes=False),
        out_type=jax.ShapeDtypeStruct((F, N), jnp.float32),
        scratch_types=[
            pltpu.VMEM((N,), jnp.float32),
            pltpu.VMEM((N,), jnp.float32),
            pltpu.VMEM((W,), jnp.int32),
            pltpu.VMEM((cpw, W), jnp.float32),
            pltpu.SMEM((1,), jnp.int32),
        ],
    )
    def k(mT_hbm, dst_hbm, out_hbm, acc0, acc1, dstv, vals, flag):
        wid = lax.axis_index("s") * 2 + lax.axis_index("c")
        accs = [acc0, acc1][:cpw]

        def ibody(i, _):
            neg = jnp.full((16,), -jnp.inf, jnp.float32)
            for c in range(cpw):
                accs[c][pl.ds(i * 16, 16)] = neg
            return ()

        lax.fori_loop(0, N // 16, ibody, (), unroll=False)

        @pl.when(wid < nact)
        def _():
            def wbody(w, _):
                off = w * W
                pltpu.sync_copy(dst_hbm.at[pl.ds(off, W)], dstv)
                for c in range(cpw):
                    pltpu.sync_copy(mT_hbm.at[wid * cpw + c, pl.ds(off, W)],
                                    vals.at[c])

                def vpass(j, p_acc):
                    idx = dstv[pl.ds(j * 16, 16)]
                    pend = jnp.zeros((16,), jnp.bool_)
                    for c in range(cpw):
                        v = vals[c, pl.ds(j * 16, 16)]
                        chk = plsc.load_gather(accs[c], [idx])
                        p = v > chk
                        plsc.store_scatter(accs[c], [idx], v, mask=p)
                        chk2 = plsc.load_gather(accs[c], [idx])
                        pend = jnp.logical_or(pend, v > chk2)
                    return jnp.logical_or(p_acc, pend)

                pendv = lax.fori_loop(0, nv, vpass,
                                      jnp.zeros((16,), jnp.bool_),
                                      unroll=False)
                flag[0] = jnp.sum(pendv.astype(jnp.int32))

                for _ in range(15):
                    @pl.when(flag[0] > 0)
                    def _():
                        pv = lax.fori_loop(0, nv, vpass,
                                           jnp.zeros((16,), jnp.bool_),
                                           unroll=False)
                        flag[0] = jnp.sum(pv.astype(jnp.int32))

                return ()

            lax.fori_loop(0, nwin, wbody, (), unroll=False)

            for c in range(cpw):
                pltpu.sync_copy(accs[c], out_hbm.at[wid * cpw + c])

    return k


def _tfix_body(i_ref, o_ref):
    t = i_ref[...].T
    o_ref[...] = jnp.where(t == -jnp.inf, 0.0, t)


@functools.lru_cache(maxsize=None)
def _make_tfix(F, N):
    return pl.pallas_call(
        _tfix_body,
        out_shape=jax.ShapeDtypeStruct((N, F), jnp.float32),
    )


def _final_body(hs_ref, hd_ref, e_ref, st_ref, gb_ref, w0, b0, w1, b1,
                w2, b2, w3, b3, w4, b4, o_ref):
    mu = st_ref[0, :]
    var = st_ref[1, :]
    g = gb_ref[0, :]
    b = gb_ref[1, :]
    en = g * (e_ref[...] - mu) / jnp.sqrt(var + 1e-5) + b
    hcat = jnp.concatenate([hs_ref[...], hd_ref[...], en], axis=1)
    h = jnp.dot(hcat, w0[...].T, preferred_element_type=jnp.float32)
    h = _lrelu(h + b0[...])
    h = _lrelu(jnp.dot(h, w1[...].T, preferred_element_type=jnp.float32) + b1[...])
    h = _lrelu(jnp.dot(h, w2[...].T, preferred_element_type=jnp.float32) + b2[...])
    h = _lrelu(jnp.dot(h, w3[...].T, preferred_element_type=jnp.float32) + b3[...])
    o_ref[...] = jnp.dot(h, w4[...].T, preferred_element_type=jnp.float32) + b4[...]


@functools.lru_cache(maxsize=None)
def _make_final(E):
    shapes = [(64,), (32, 64), (32,), (16, 32), (16,),
              (8, 16), (8,), (2, 8), (2,)]
    specs = [
        pl.BlockSpec((BE, 64), lambda i: (i, 0)),
        pl.BlockSpec((BE, 64), lambda i: (i, 0)),
        pl.BlockSpec((BE, 10), lambda i: (i, 0)),
        pl.BlockSpec((2, 10), lambda i: (0, 0)),
        pl.BlockSpec((2, 10), lambda i: (0, 0)),
        pl.BlockSpec((64, 138), lambda i: (0, 0)),
    ]
    for s in shapes:
        if len(s) == 2:
            specs.append(pl.BlockSpec(s, lambda i: (0, 0)))
        else:
            specs.append(pl.BlockSpec(s, lambda i: (0,)))
    return pl.pallas_call(
        _final_body,
        grid=(E // BE,),
        in_specs=specs,
        out_specs=pl.BlockSpec((BE, 2), lambda i: (i, 0)),
        out_shape=jax.ShapeDtypeStruct((E, 2), jnp.float32),
    )


def _edge_conv(x, src, dst, w0, b0, w1, b1, w2, b2, n):
    E = src.shape[0]
    F = x.shape[1]
    H0, H1, F2 = w0.shape[0], w1.shape[0], w2.shape[0]
    xj, xi = _pair_gather(x, src, dst)
    mT = _make_mlp3T(E, F, H0, H1, F2)(xi, xj, w0, b0, w1, b1, w2, b2)
    oT = _make_scatter_max(n, F2, E)(mT, dst)
    return _make_tfix(F2, n)(oT)


def kernel(x, edge_index, e, xbatch, params):
    p = params
    src = edge_index[0]
    dst = edge_index[1]
    n = x.shape[0]
    E = src.shape[0]

    stx = _bn_stats(x, 10000)
    ste = _bn_stats(e, 8000)
    gbx = jnp.stack([p['bn_node_g'], p['bn_node_b']])
    gbe = jnp.stack([p['bn_edge_g'], p['bn_edge_b']])

    xn = _make_bnx(n, 16)(x, stx, gbx)
    x1 = _edge_conv(xn, src, dst, p['nn0_w0'], p['nn0_b0'],
                    p['nn0_w1'], p['nn0_b1'], p['nn0_w2'], p['nn0_b2'], n)
    x2 = _edge_conv(x1, src, dst, p['nn1_w0'], p['nn1_b0'],
                    p['nn1_w1'], p['nn1_b1'], p['nn1_w2'], p['nn1_b2'], n)
    x3 = _edge_conv(x2, src, dst, p['nn2_w0'], p['nn2_b0'],
                    p['nn2_w1'], p['nn2_b1'], p['nn2_w2'], p['nn2_b2'], n)

    hs, hd = _pair_gather(x3, src, dst)
    return _make_final(E)(
        hs, hd, e, ste, gbe, p['ep_w0'], p['ep_b0'],
        p['ep_w1'], p['ep_b1'], p['ep_w2'], p['ep_b2'],
        p['ep_w3'], p['ep_b3'], p['ep_w4'], p['ep_b4'])
